# baseline (speedup 1.0000x reference)
import functools

import jax
import jax.numpy as jnp
from jax import lax
from jax.experimental import pallas as pl
from jax.experimental.pallas import tpu as pltpu
from jax.experimental.pallas import tpu_sc as plsc

_ROWS = 8192
_COLS = 1024
_SC_ROWS = 2048
_TC_ROWS = _ROWS - _SC_ROWS
_NWORKERS = 32
_ROWS_PER_W = _SC_ROWS // _NWORKERS
_CHUNK_ROWS = 32
_NCHUNKS = _ROWS_PER_W // _CHUNK_ROWS
_NBUF = 2

_mesh = plsc.VectorSubcoreMesh(core_axis_name="c", subcore_axis_name="s")


@functools.partial(
    pl.kernel,
    out_type=jax.ShapeDtypeStruct((_SC_ROWS, _COLS), jnp.float32),
    mesh=_mesh,
    scratch_types=[
        pltpu.VMEM((_NBUF, _CHUNK_ROWS, _COLS), jnp.float32),
        pltpu.SemaphoreType.DMA((_NBUF,)),
        pltpu.SemaphoreType.DMA((_NBUF,)),
    ],
)
def _sc_copy(x_hbm, o_hbm, bufs, load_sems, store_sems):
    wid = lax.axis_index("s") * 2 + lax.axis_index("c")
    base = wid * _ROWS_PER_W

    def load(i, b):
        return pltpu.make_async_copy(
            x_hbm.at[pl.ds(base + i * _CHUNK_ROWS, _CHUNK_ROWS), :],
            bufs.at[b],
            load_sems.at[b],
        )

    def store(i, b):
        return pltpu.make_async_copy(
            bufs.at[b],
            o_hbm.at[pl.ds(base + i * _CHUNK_ROWS, _CHUNK_ROWS), :],
            store_sems.at[b],
        )

    for i in range(_NBUF):
        load(i, i).start()
    for i in range(_NCHUNKS):
        b = i % _NBUF
        load(i, b).wait()
        store(i, b).start()
        nxt = i + _NBUF
        if nxt < _NCHUNKS:
            store(nxt - _NBUF, b).wait()
            load(nxt, b).start()
    for i in range(_NCHUNKS - _NBUF, _NCHUNKS):
        store(i, i % _NBUF).wait()


def _tc_copy_block(x_ref, o_ref):
    o_ref[...] = x_ref[...]


def kernel(x):
    sc_part = _sc_copy(x[:_SC_ROWS])
    tc_part = pl.pallas_call(
        _tc_copy_block,
        grid=(3,),
        in_specs=[pl.BlockSpec((2048, _COLS), lambda i: (i, 0))],
        out_specs=pl.BlockSpec((2048, _COLS), lambda i: (i, 0)),
        out_shape=jax.ShapeDtypeStruct((_TC_ROWS, _COLS), x.dtype),
    )(x[_SC_ROWS:])
    gathered = jnp.concatenate([sc_part, tc_part], axis=0)
    sizes = jnp.array([x.shape[0]], dtype=jnp.int64)
    return (gathered, sizes)


# speedup vs baseline: 3.7394x; 3.7394x over previous
import jax
import jax.numpy as jnp
from jax.experimental import pallas as pl
from jax.experimental.pallas import tpu as pltpu

_NBUF = 8
_CHUNK_ROWS = 1024


def _dma_ring(x_hbm, o_hbm, bufs, load_sems, store_sems):
    nchunks = x_hbm.shape[0] // _CHUNK_ROWS

    def load(i, b):
        return pltpu.make_async_copy(
            x_hbm.at[pl.ds(i * _CHUNK_ROWS, _CHUNK_ROWS), :],
            bufs.at[b],
            load_sems.at[b],
        )

    def store(i, b):
        return pltpu.make_async_copy(
            bufs.at[b],
            o_hbm.at[pl.ds(i * _CHUNK_ROWS, _CHUNK_ROWS), :],
            store_sems.at[b],
        )

    for i in range(min(_NBUF, nchunks)):
        load(i, i).start()
    for i in range(nchunks):
        b = i % _NBUF
        load(i, b).wait()
        store(i, b).start()
        nxt = i + _NBUF
        if nxt < nchunks:
            store(nxt - _NBUF, b).wait()
            load(nxt, b).start()
    for i in range(max(nchunks - _NBUF, 0), nchunks):
        store(i, i % _NBUF).wait()


def kernel(x):
    rows, cols = x.shape
    gathered = pl.pallas_call(
        _dma_ring,
        in_specs=[pl.BlockSpec(memory_space=pl.ANY)],
        out_specs=pl.BlockSpec(memory_space=pl.ANY),
        out_shape=jax.ShapeDtypeStruct((rows, cols), x.dtype),
        scratch_shapes=[
            pltpu.VMEM((_NBUF, _CHUNK_ROWS, cols), x.dtype),
            pltpu.SemaphoreType.DMA((_NBUF,)),
            pltpu.SemaphoreType.DMA((_NBUF,)),
        ],
    )(x)
    sizes = jnp.array([rows], dtype=jnp.int64)
    return (gathered, sizes)


# speedup vs baseline: 3.8791x; 1.0374x over previous
import jax
import jax.numpy as jnp
from jax.experimental import pallas as pl
from jax.experimental.pallas import tpu as pltpu

_CHUNKS = (256, 256, 512, 1024, 1024, 2048, 2048, 1024)


def _dma_ring(x_hbm, o_hbm, buf, load_sems, store_sems):
    offs = []
    off = 0
    for c in _CHUNKS:
        offs.append(off)
        off += c

    def load(k):
        return pltpu.make_async_copy(
            x_hbm.at[pl.ds(offs[k], _CHUNKS[k]), :],
            buf.at[pl.ds(offs[k], _CHUNKS[k]), :],
            load_sems.at[k],
        )

    def store(k):
        return pltpu.make_async_copy(
            buf.at[pl.ds(offs[k], _CHUNKS[k]), :],
            o_hbm.at[pl.ds(offs[k], _CHUNKS[k]), :],
            store_sems.at[k],
        )

    for k in range(len(_CHUNKS)):
        load(k).start()
    for k in range(len(_CHUNKS)):
        load(k).wait()
        store(k).start()
    for k in range(len(_CHUNKS)):
        store(k).wait()


def kernel(x):
    rows, cols = x.shape
    nk = len(_CHUNKS)
    gathered = pl.pallas_call(
        _dma_ring,
        in_specs=[pl.BlockSpec(memory_space=pl.ANY)],
        out_specs=pl.BlockSpec(memory_space=pl.ANY),
        out_shape=jax.ShapeDtypeStruct((rows, cols), x.dtype),
        scratch_shapes=[
            pltpu.VMEM((rows, cols), x.dtype),
            pltpu.SemaphoreType.DMA((nk,)),
            pltpu.SemaphoreType.DMA((nk,)),
        ],
    )(x)
    sizes = jnp.array([rows], dtype=jnp.int64)
    return (gathered, sizes)
